# baseline (speedup 1.0000x reference)
import jax
import jax.numpy as jnp
from jax import lax
from jax.experimental import pallas as pl
from jax.experimental.pallas import tpu as pltpu

_KEY_DIM = 256
_PFC_DIM = 32
_N_SLOTS = 8192
_EPS = 8
_D_MEM = 44
_BLK = 1024
_NBLK = _N_SLOTS // _BLK
_GLOBAL_STEP = 100.0


def _body(proto_ref, act_ref, pfc_ref, ctd_ref, w1a_ref, w1b_ref, b1_ref,
          w2t_ref, b2_ref, scw1t_ref, scb1_ref, scw2_ref, scb2_ref,
          gw1t_ref, gb1_ref, gw2_ref, gb2_ref, rpwt_ref, rpb_ref,
          rnwt_ref, rnb_ref, ep_hbm, td_hbm, ts_hbm,
          o_pfc, o_alpha, o_nm, o_onehot,
          sims_s, kn_s, ep_s, td_s, ts_s, sem):
    i = pl.program_id(0)

    @pl.when(i == 0)
    def _make_key():
        act = act_ref[...]
        pfc = pfc_ref[...]
        h = lax.dot_general(act, w1a_ref[...], (((1,), (0,)), ((), ())),
                            preferred_element_type=jnp.float32)
        h = h + lax.dot_general(pfc, w1b_ref[...], (((1,), (0,)), ((), ())),
                                preferred_element_type=jnp.float32)
        h = jnp.maximum(h + b1_ref[...], 0.0)
        key = lax.dot_general(h, w2t_ref[...], (((1,), (0,)), ((), ())),
                              preferred_element_type=jnp.float32)
        key = key + b2_ref[...]
        kn = jnp.sqrt(jnp.sum(key * key, axis=1, keepdims=True))
        kn_s[...] = key / jnp.maximum(kn, 1e-12)

    kn = kn_s[...]
    blk = proto_ref[...]
    dots = lax.dot_general(kn, blk, (((1,), (1,)), ((), ())),
                           preferred_element_type=jnp.float32)
    ones = jnp.ones((1, _KEY_DIM), jnp.float32)
    n2 = lax.dot_general(ones, blk * blk, (((1,), (1,)), ((), ())),
                         preferred_element_type=jnp.float32)
    sims_row = dots / jnp.maximum(jnp.sqrt(n2), 1e-12)
    sims_s[pl.ds(i, 1), :] = sims_row

    @pl.when(i == _NBLK - 1)
    def _tail():
        sims = sims_s[...]
        best_sim = jnp.max(sims)
        flat = (lax.broadcasted_iota(jnp.int32, (_NBLK, _BLK), 0) * _BLK
                + lax.broadcasted_iota(jnp.int32, (_NBLK, _BLK), 1))
        slot = jnp.min(jnp.where(sims == best_sim, flat, jnp.int32(2**30)))

        gi = (lax.broadcasted_iota(jnp.int32, (64, 128), 0) * 128
              + lax.broadcasted_iota(jnp.int32, (64, 128), 1))
        o_onehot[...] = (gi == slot).astype(jnp.float32)

        cp0 = pltpu.make_async_copy(ep_hbm.at[slot], ep_s, sem.at[0])
        cp1 = pltpu.make_async_copy(td_hbm.at[pl.ds(slot, 1), :], td_s,
                                    sem.at[1])
        cp2 = pltpu.make_async_copy(ts_hbm.at[pl.ds(slot, 1), :], ts_s,
                                    sem.at[2])
        cp0.start(); cp1.start(); cp2.start()
        cp0.wait(); cp1.wait(); cp2.wait()

        ep = ep_s[...]
        stored = ep[:, :_PFC_DIM]
        pfc = pfc_ref[...]
        pfc_n = pfc / jnp.maximum(
            jnp.sqrt(jnp.sum(pfc * pfc, axis=1, keepdims=True)), 1e-12)
        sn = jnp.sqrt(jnp.sum(stored * stored, axis=1, keepdims=True))
        stored_n = stored / jnp.maximum(sn, 1e-12)
        ep_sims = jnp.sum(stored_n * pfc_n, axis=1, keepdims=True)

        td_row = td_s[...]
        ts_row = ts_s[...]
        ages = _GLOBAL_STEP - ts_row
        max_age = jnp.maximum(jnp.max(ages), 1.0)
        rec_row = 1.0 - ages / max_age

        r8 = lax.broadcasted_iota(jnp.int32, (_EPS, _EPS), 0)
        c8 = lax.broadcasted_iota(jnp.int32, (_EPS, _EPS), 1)
        eye = r8 == c8
        zero8 = jnp.zeros((_EPS, _EPS), jnp.float32)
        td_col = jnp.sum(jnp.where(eye, td_row + zero8, zero8),
                         axis=1, keepdims=True)
        rec_col = jnp.sum(jnp.where(eye, rec_row + zero8, zero8),
                          axis=1, keepdims=True)
        f_td = jnp.maximum(jnp.abs(td_col), 1e-6)

        hs = jnp.maximum(
            ep_sims * scw1t_ref[0:1, :] + f_td * scw1t_ref[1:2, :]
            + rec_col * scw1t_ref[2:3, :] + scb1_ref[...], 0.0)
        rel = (jnp.sum(hs * scw2_ref[...], axis=1, keepdims=True)
               + scb2_ref[0, 0])
        mrel = jnp.max(rel)
        eidx = lax.broadcasted_iota(jnp.int32, (_EPS, 1), 0)
        bidx = jnp.min(jnp.where(rel == mrel, eidx, jnp.int32(2**30)))
        sel = eidx == bidx
        ep_content = jnp.sum(jnp.where(sel, ep, 0.0), axis=0,
                             keepdims=True)
        ep_td = jnp.sum(jnp.where(sel, td_col, 0.0))

        ctd = jnp.abs(ctd_ref[0, 0])
        hg = jnp.tanh(best_sim * gw1t_ref[0:1, :] + ctd * gw1t_ref[1:2, :]
                      + ep_td * gw1t_ref[2:3, :] + gb1_ref[...])
        alpha = jnp.tanh(jnp.sum(hg * gw2_ref[...]) + gb2_ref[0, 0])
        o_alpha[...] = alpha * jnp.ones((1, 1), jnp.float32)

        delta = lax.dot_general(ep_content, rpwt_ref[...],
                                (((1,), (0,)), ((), ())),
                                preferred_element_type=jnp.float32)
        o_pfc[...] = pfc + alpha * (delta + rpb_ref[...])

        nm = lax.dot_general(ep_content, rnwt_ref[...],
                             (((1,), (0,)), ((), ())),
                             preferred_element_type=jnp.float32)
        nm = nm + rnb_ref[...]
        lane = lax.broadcasted_iota(jnp.int32, (1, 12), 1)
        hi = jnp.where(lane < 8, 1.0, 0.5)
        o_nm[...] = jnp.clip(nm, 0.1, hi)


def kernel(activation_summary, pfc_state, current_td_error, prototypes,
           log_temperature, kp_w1, kp_b1, kp_w2, kp_b2, episodes,
           ep_td_errors, ep_timestamps, sc_w1, sc_b1, sc_w2, sc_b2,
           g_w1, g_b1, g_w2, g_b2, rp_w, rp_b, rn_w, rn_b):
    del log_temperature
    act = activation_summary.reshape(1, _KEY_DIM)
    ctd = current_td_error.reshape(1, 1)
    w1a = jnp.transpose(kp_w1[:, :_KEY_DIM])
    w1b = jnp.transpose(kp_w1[:, _KEY_DIM:])
    b1 = kp_b1.reshape(1, -1)
    w2t = jnp.transpose(kp_w2)
    b2 = kp_b2.reshape(1, -1)
    scw1t = jnp.transpose(sc_w1)
    scb1 = sc_b1.reshape(1, -1)
    scb2 = sc_b2.reshape(1, 1)
    gw1t = jnp.transpose(g_w1)
    gb1 = g_b1.reshape(1, -1)
    gb2 = g_b2.reshape(1, 1)
    rpwt = jnp.transpose(rp_w)
    rpb = rp_b.reshape(1, -1)
    rnwt = jnp.transpose(rn_w)
    rnb = rn_b.reshape(1, -1)

    full = lambda shape: pl.BlockSpec(shape, lambda i: (0,) * len(shape))
    outs = pl.pallas_call(
        _body,
        grid=(_NBLK,),
        in_specs=[
            pl.BlockSpec((_BLK, _KEY_DIM), lambda i: (i, 0)),
            full((1, _KEY_DIM)), full((1, _PFC_DIM)), full((1, 1)),
            full((_KEY_DIM, 512)), full((_PFC_DIM, 512)), full((1, 512)),
            full((512, _KEY_DIM)), full((1, _KEY_DIM)),
            full((3, 8)), full((1, 8)), full((1, 8)), full((1, 1)),
            full((3, 16)), full((1, 16)), full((1, 16)), full((1, 1)),
            full((_D_MEM, _PFC_DIM)), full((1, _PFC_DIM)),
            full((_D_MEM, 12)), full((1, 12)),
            pl.BlockSpec(memory_space=pl.ANY),
            pl.BlockSpec(memory_space=pl.ANY),
            pl.BlockSpec(memory_space=pl.ANY),
        ],
        out_specs=[full((1, _PFC_DIM)), full((1, 1)), full((1, 12)),
                   full((64, 128))],
        out_shape=[
            jax.ShapeDtypeStruct((1, _PFC_DIM), jnp.float32),
            jax.ShapeDtypeStruct((1, 1), jnp.float32),
            jax.ShapeDtypeStruct((1, 12), jnp.float32),
            jax.ShapeDtypeStruct((64, 128), jnp.float32),
        ],
        scratch_shapes=[
            pltpu.VMEM((_NBLK, _BLK), jnp.float32),
            pltpu.VMEM((1, _KEY_DIM), jnp.float32),
            pltpu.VMEM((_EPS, _D_MEM), jnp.float32),
            pltpu.VMEM((1, _EPS), jnp.float32),
            pltpu.VMEM((1, _EPS), jnp.float32),
            pltpu.SemaphoreType.DMA((3,)),
        ],
        compiler_params=pltpu.CompilerParams(
            dimension_semantics=("arbitrary",)),
    )(prototypes, act, pfc_state, ctd, w1a, w1b, b1, w2t, b2,
      scw1t, scb1, sc_w2, scb2, gw1t, gb1, g_w2, gb2, rpwt, rpb,
      rnwt, rnb, episodes, ep_td_errors, ep_timestamps)

    o_pfc, o_alpha, o_nm, o_onehot = outs
    return jnp.concatenate([o_pfc.reshape(_PFC_DIM), o_alpha.reshape(1),
                            o_onehot.reshape(_N_SLOTS), o_nm.reshape(12)])


# speedup vs baseline: 1.0537x; 1.0537x over previous
import jax
import jax.numpy as jnp
from jax import lax
from jax.experimental import pallas as pl
from jax.experimental.pallas import tpu as pltpu

_KEY_DIM = 256
_PFC_DIM = 32
_N_SLOTS = 8192
_EPS = 8
_D_MEM = 44
_BLK = 1024
_NBLK = _N_SLOTS // _BLK
_GLOBAL_STEP = 100.0

_DNT = (((1,), (1,)), ((), ()))


def _body(proto_ref, act_ref, pfc_ref, ctd_ref, w1_ref, b1_ref,
          w2_ref, b2_ref, scw1_ref, scb1_ref, scw2_ref, scb2_ref,
          gw1_ref, gb1_ref, gw2_ref, gb2_ref, rpw_ref, rpb_ref,
          rnw_ref, rnb_ref, ep_hbm, td_hbm, ts_hbm,
          o_pfc, o_alpha, o_nm, o_onehot,
          sims_s, kn_s, ep_s, td_s, ts_s, sem):
    i = pl.program_id(0)

    @pl.when(i == 0)
    def _make_key():
        act = act_ref[...]
        pfc = pfc_ref[...]
        w1 = w1_ref[...]
        h = lax.dot_general(act, w1[:, :_KEY_DIM], _DNT,
                            preferred_element_type=jnp.float32)
        h = h + lax.dot_general(pfc, w1[:, _KEY_DIM:], _DNT,
                                preferred_element_type=jnp.float32)
        h = jnp.maximum(h + b1_ref[...], 0.0)
        key = lax.dot_general(h, w2_ref[...], _DNT,
                              preferred_element_type=jnp.float32)
        key = key + b2_ref[...]
        kn = jnp.sqrt(jnp.sum(key * key, axis=1, keepdims=True))
        kn_s[...] = key / jnp.maximum(kn, 1e-12)

    kn = kn_s[...]
    blk = proto_ref[...]
    dots = lax.dot_general(kn, blk, _DNT,
                           preferred_element_type=jnp.float32)
    ones = jnp.ones((1, _KEY_DIM), jnp.float32)
    n2 = lax.dot_general(ones, blk * blk, _DNT,
                         preferred_element_type=jnp.float32)
    sims_row = dots / jnp.maximum(jnp.sqrt(n2), 1e-12)
    sims_s[pl.ds(i, 1), :] = sims_row

    @pl.when(i == _NBLK - 1)
    def _tail():
        sims = sims_s[...]
        best_sim = jnp.max(sims)
        flat = (lax.broadcasted_iota(jnp.int32, (_NBLK, _BLK), 0) * _BLK
                + lax.broadcasted_iota(jnp.int32, (_NBLK, _BLK), 1))
        slot = jnp.min(jnp.where(sims == best_sim, flat, jnp.int32(2**30)))

        gi = (lax.broadcasted_iota(jnp.int32, (64, 128), 0) * 128
              + lax.broadcasted_iota(jnp.int32, (64, 128), 1))
        o_onehot[...] = (gi == slot).astype(jnp.float32)

        cp0 = pltpu.make_async_copy(ep_hbm.at[slot], ep_s, sem.at[0])
        cp1 = pltpu.make_async_copy(td_hbm.at[pl.ds(slot, 1), :], td_s,
                                    sem.at[1])
        cp2 = pltpu.make_async_copy(ts_hbm.at[pl.ds(slot, 1), :], ts_s,
                                    sem.at[2])
        cp0.start(); cp1.start(); cp2.start()
        cp0.wait(); cp1.wait(); cp2.wait()

        ep = ep_s[...]
        stored = ep[:, :_PFC_DIM]
        pfc = pfc_ref[...]
        pfc_n = pfc / jnp.maximum(
            jnp.sqrt(jnp.sum(pfc * pfc, axis=1, keepdims=True)), 1e-12)
        sn = jnp.sqrt(jnp.sum(stored * stored, axis=1, keepdims=True))
        stored_n = stored / jnp.maximum(sn, 1e-12)
        ep_sims = jnp.sum(stored_n * pfc_n, axis=1, keepdims=True)

        td_row = td_s[...]
        ts_row = ts_s[...]
        ages = _GLOBAL_STEP - ts_row
        max_age = jnp.maximum(jnp.max(ages), 1.0)
        rec_row = 1.0 - ages / max_age

        r8 = lax.broadcasted_iota(jnp.int32, (_EPS, _EPS), 0)
        c8 = lax.broadcasted_iota(jnp.int32, (_EPS, _EPS), 1)
        eye = r8 == c8
        zero8 = jnp.zeros((_EPS, _EPS), jnp.float32)
        td_col = jnp.sum(jnp.where(eye, td_row + zero8, zero8),
                         axis=1, keepdims=True)
        rec_col = jnp.sum(jnp.where(eye, rec_row + zero8, zero8),
                          axis=1, keepdims=True)
        f_td = jnp.maximum(jnp.abs(td_col), 1e-6)

        lane3 = lax.broadcasted_iota(jnp.int32, (_EPS, 3), 1)
        zero3 = jnp.zeros((_EPS, 3), jnp.float32)
        scorer_in = jnp.where(lane3 == 0, ep_sims + zero3,
                              jnp.where(lane3 == 1, f_td + zero3,
                                        rec_col + zero3))
        hs = jnp.maximum(
            lax.dot_general(scorer_in, scw1_ref[...], _DNT,
                            preferred_element_type=jnp.float32)
            + scb1_ref[...], 0.0)
        rel = (jnp.sum(hs * scw2_ref[...], axis=1, keepdims=True)
               + scb2_ref[...])
        mrel = jnp.max(rel)
        eidx = lax.broadcasted_iota(jnp.int32, (_EPS, 1), 0)
        bidx = jnp.min(jnp.where(rel == mrel, eidx, jnp.int32(2**30)))
        sel = eidx == bidx
        ep_content = jnp.sum(jnp.where(sel, ep, 0.0), axis=0,
                             keepdims=True)
        ep_td = jnp.sum(jnp.where(sel, td_col, 0.0))

        ctd = jnp.abs(ctd_ref[0, 0])
        glane = lax.broadcasted_iota(jnp.int32, (1, 3), 1)
        gzero = jnp.zeros((1, 3), jnp.float32)
        gate_in = jnp.where(glane == 0, best_sim + gzero,
                            jnp.where(glane == 1, ctd + gzero,
                                      ep_td + gzero))
        hg = jnp.tanh(lax.dot_general(gate_in, gw1_ref[...], _DNT,
                                      preferred_element_type=jnp.float32)
                      + gb1_ref[...])
        alpha = jnp.tanh(jnp.sum(hg * gw2_ref[...]) + gb2_ref[0, 0])
        o_alpha[...] = alpha * jnp.ones((1, 1), jnp.float32)

        delta = lax.dot_general(ep_content, rpw_ref[...], _DNT,
                                preferred_element_type=jnp.float32)
        o_pfc[...] = pfc + alpha * (delta + rpb_ref[...])

        nm = lax.dot_general(ep_content, rnw_ref[...], _DNT,
                             preferred_element_type=jnp.float32)
        nm = nm + rnb_ref[...]
        lane = lax.broadcasted_iota(jnp.int32, (1, 12), 1)
        hi = jnp.where(lane < 8, 1.0, 0.5)
        o_nm[...] = jnp.clip(nm, 0.1, hi)


def kernel(activation_summary, pfc_state, current_td_error, prototypes,
           log_temperature, kp_w1, kp_b1, kp_w2, kp_b2, episodes,
           ep_td_errors, ep_timestamps, sc_w1, sc_b1, sc_w2, sc_b2,
           g_w1, g_b1, g_w2, g_b2, rp_w, rp_b, rn_w, rn_b):
    del log_temperature
    act = activation_summary.reshape(1, _KEY_DIM)
    ctd = current_td_error.reshape(1, 1)

    full = lambda shape: pl.BlockSpec(shape, lambda i: (0,) * len(shape))
    outs = pl.pallas_call(
        _body,
        grid=(_NBLK,),
        in_specs=[
            pl.BlockSpec((_BLK, _KEY_DIM), lambda i: (i, 0)),
            full((1, _KEY_DIM)), full((1, _PFC_DIM)), full((1, 1)),
            full((512, _KEY_DIM + _PFC_DIM)), full((1, 512)),
            full((_KEY_DIM, 512)), full((1, _KEY_DIM)),
            full((8, 3)), full((1, 8)), full((1, 8)), full((1, 1)),
            full((16, 3)), full((1, 16)), full((1, 16)), full((1, 1)),
            full((_PFC_DIM, _D_MEM)), full((1, _PFC_DIM)),
            full((12, _D_MEM)), full((1, 12)),
            pl.BlockSpec(memory_space=pl.ANY),
            pl.BlockSpec(memory_space=pl.ANY),
            pl.BlockSpec(memory_space=pl.ANY),
        ],
        out_specs=[full((1, _PFC_DIM)), full((1, 1)), full((1, 12)),
                   full((64, 128))],
        out_shape=[
            jax.ShapeDtypeStruct((1, _PFC_DIM), jnp.float32),
            jax.ShapeDtypeStruct((1, 1), jnp.float32),
            jax.ShapeDtypeStruct((1, 12), jnp.float32),
            jax.ShapeDtypeStruct((64, 128), jnp.float32),
        ],
        scratch_shapes=[
            pltpu.VMEM((_NBLK, _BLK), jnp.float32),
            pltpu.VMEM((1, _KEY_DIM), jnp.float32),
            pltpu.VMEM((_EPS, _D_MEM), jnp.float32),
            pltpu.VMEM((1, _EPS), jnp.float32),
            pltpu.VMEM((1, _EPS), jnp.float32),
            pltpu.SemaphoreType.DMA((3,)),
        ],
        compiler_params=pltpu.CompilerParams(
            dimension_semantics=("arbitrary",)),
    )(prototypes, act, pfc_state, ctd, kp_w1, kp_b1.reshape(1, -1),
      kp_w2, kp_b2.reshape(1, -1), sc_w1, sc_b1.reshape(1, -1),
      sc_w2, sc_b2.reshape(1, 1), g_w1, g_b1.reshape(1, -1),
      g_w2, g_b2.reshape(1, 1), rp_w, rp_b.reshape(1, -1),
      rn_w, rn_b.reshape(1, -1), episodes, ep_td_errors, ep_timestamps)

    o_pfc, o_alpha, o_nm, o_onehot = outs
    return jnp.concatenate([o_pfc.reshape(_PFC_DIM), o_alpha.reshape(1),
                            o_onehot.reshape(_N_SLOTS), o_nm.reshape(12)])


# speedup vs baseline: 1.1022x; 1.0460x over previous
import jax
import jax.numpy as jnp
from jax import lax
from jax.experimental import pallas as pl
from jax.experimental.pallas import tpu as pltpu

_KEY_DIM = 256
_PFC_DIM = 32
_N_SLOTS = 8192
_EPS = 8
_D_MEM = 44
_BLK = 8192
_NBLK = _N_SLOTS // _BLK
_GLOBAL_STEP = 100.0

_DNT = (((1,), (1,)), ((), ()))


def _body(proto_ref, act_ref, pfc_ref, ctd_ref, w1_ref, b1_ref,
          w2_ref, b2_ref, scw1_ref, scb1_ref, scw2_ref, scb2_ref,
          gw1_ref, gb1_ref, gw2_ref, gb2_ref, rpw_ref, rpb_ref,
          rnw_ref, rnb_ref, ep_hbm, td_hbm, ts_hbm,
          o_pfc, o_alpha, o_nm, o_onehot,
          sims_s, kn_s, ep_s, td_s, ts_s, sem):
    i = pl.program_id(0)

    @pl.when(i == 0)
    def _make_key():
        act = act_ref[...]
        pfc = pfc_ref[...]
        w1 = w1_ref[...]
        h = lax.dot_general(act, w1[:, :_KEY_DIM], _DNT,
                            preferred_element_type=jnp.float32)
        h = h + lax.dot_general(pfc, w1[:, _KEY_DIM:], _DNT,
                                preferred_element_type=jnp.float32)
        h = jnp.maximum(h + b1_ref[...], 0.0)
        key = lax.dot_general(h, w2_ref[...], _DNT,
                              preferred_element_type=jnp.float32)
        key = key + b2_ref[...]
        kn = jnp.sqrt(jnp.sum(key * key, axis=1, keepdims=True))
        kn_s[...] = key / jnp.maximum(kn, 1e-12)

    kn = kn_s[...]
    blk = proto_ref[...]
    dots = lax.dot_general(kn, blk, _DNT,
                           preferred_element_type=jnp.float32)
    ones = jnp.ones((1, _KEY_DIM), jnp.float32)
    n2 = lax.dot_general(ones, blk * blk, _DNT,
                         preferred_element_type=jnp.float32)
    sims_row = dots / jnp.maximum(jnp.sqrt(n2), 1e-12)
    sims_s[pl.ds(i, 1), :] = sims_row

    @pl.when(i == _NBLK - 1)
    def _tail():
        sims = sims_s[...]
        best_sim = jnp.max(sims)
        flat = (lax.broadcasted_iota(jnp.int32, (_NBLK, _BLK), 0) * _BLK
                + lax.broadcasted_iota(jnp.int32, (_NBLK, _BLK), 1))
        slot = jnp.min(jnp.where(sims == best_sim, flat, jnp.int32(2**30)))

        gi = (lax.broadcasted_iota(jnp.int32, (64, 128), 0) * 128
              + lax.broadcasted_iota(jnp.int32, (64, 128), 1))
        o_onehot[...] = (gi == slot).astype(jnp.float32)

        cp0 = pltpu.make_async_copy(ep_hbm.at[slot], ep_s, sem.at[0])
        cp1 = pltpu.make_async_copy(td_hbm.at[pl.ds(slot, 1), :], td_s,
                                    sem.at[1])
        cp2 = pltpu.make_async_copy(ts_hbm.at[pl.ds(slot, 1), :], ts_s,
                                    sem.at[2])
        cp0.start(); cp1.start(); cp2.start()
        cp0.wait(); cp1.wait(); cp2.wait()

        ep = ep_s[...]
        stored = ep[:, :_PFC_DIM]
        pfc = pfc_ref[...]
        pfc_n = pfc / jnp.maximum(
            jnp.sqrt(jnp.sum(pfc * pfc, axis=1, keepdims=True)), 1e-12)
        sn = jnp.sqrt(jnp.sum(stored * stored, axis=1, keepdims=True))
        stored_n = stored / jnp.maximum(sn, 1e-12)
        ep_sims = jnp.sum(stored_n * pfc_n, axis=1, keepdims=True)

        td_row = td_s[...]
        ts_row = ts_s[...]
        ages = _GLOBAL_STEP - ts_row
        max_age = jnp.maximum(jnp.max(ages), 1.0)
        rec_row = 1.0 - ages / max_age

        r8 = lax.broadcasted_iota(jnp.int32, (_EPS, _EPS), 0)
        c8 = lax.broadcasted_iota(jnp.int32, (_EPS, _EPS), 1)
        eye = r8 == c8
        zero8 = jnp.zeros((_EPS, _EPS), jnp.float32)
        td_col = jnp.sum(jnp.where(eye, td_row + zero8, zero8),
                         axis=1, keepdims=True)
        rec_col = jnp.sum(jnp.where(eye, rec_row + zero8, zero8),
                          axis=1, keepdims=True)
        f_td = jnp.maximum(jnp.abs(td_col), 1e-6)

        lane3 = lax.broadcasted_iota(jnp.int32, (_EPS, 3), 1)
        zero3 = jnp.zeros((_EPS, 3), jnp.float32)
        scorer_in = jnp.where(lane3 == 0, ep_sims + zero3,
                              jnp.where(lane3 == 1, f_td + zero3,
                                        rec_col + zero3))
        hs = jnp.maximum(
            lax.dot_general(scorer_in, scw1_ref[...], _DNT,
                            preferred_element_type=jnp.float32)
            + scb1_ref[...], 0.0)
        rel = (jnp.sum(hs * scw2_ref[...], axis=1, keepdims=True)
               + scb2_ref[...])
        mrel = jnp.max(rel)
        eidx = lax.broadcasted_iota(jnp.int32, (_EPS, 1), 0)
        bidx = jnp.min(jnp.where(rel == mrel, eidx, jnp.int32(2**30)))
        sel = eidx == bidx
        ep_content = jnp.sum(jnp.where(sel, ep, 0.0), axis=0,
                             keepdims=True)
        ep_td = jnp.sum(jnp.where(sel, td_col, 0.0))

        ctd = jnp.abs(ctd_ref[0, 0])
        glane = lax.broadcasted_iota(jnp.int32, (1, 3), 1)
        gzero = jnp.zeros((1, 3), jnp.float32)
        gate_in = jnp.where(glane == 0, best_sim + gzero,
                            jnp.where(glane == 1, ctd + gzero,
                                      ep_td + gzero))
        hg = jnp.tanh(lax.dot_general(gate_in, gw1_ref[...], _DNT,
                                      preferred_element_type=jnp.float32)
                      + gb1_ref[...])
        alpha = jnp.tanh(jnp.sum(hg * gw2_ref[...]) + gb2_ref[0, 0])
        o_alpha[...] = alpha * jnp.ones((1, 1), jnp.float32)

        delta = lax.dot_general(ep_content, rpw_ref[...], _DNT,
                                preferred_element_type=jnp.float32)
        o_pfc[...] = pfc + alpha * (delta + rpb_ref[...])

        nm = lax.dot_general(ep_content, rnw_ref[...], _DNT,
                             preferred_element_type=jnp.float32)
        nm = nm + rnb_ref[...]
        lane = lax.broadcasted_iota(jnp.int32, (1, 12), 1)
        hi = jnp.where(lane < 8, 1.0, 0.5)
        o_nm[...] = jnp.clip(nm, 0.1, hi)


def kernel(activation_summary, pfc_state, current_td_error, prototypes,
           log_temperature, kp_w1, kp_b1, kp_w2, kp_b2, episodes,
           ep_td_errors, ep_timestamps, sc_w1, sc_b1, sc_w2, sc_b2,
           g_w1, g_b1, g_w2, g_b2, rp_w, rp_b, rn_w, rn_b):
    del log_temperature
    act = activation_summary.reshape(1, _KEY_DIM)
    ctd = current_td_error.reshape(1, 1)

    full = lambda shape: pl.BlockSpec(shape, lambda i: (0,) * len(shape))
    outs = pl.pallas_call(
        _body,
        grid=(_NBLK,),
        in_specs=[
            pl.BlockSpec((_BLK, _KEY_DIM), lambda i: (i, 0)),
            full((1, _KEY_DIM)), full((1, _PFC_DIM)), full((1, 1)),
            full((512, _KEY_DIM + _PFC_DIM)), full((1, 512)),
            full((_KEY_DIM, 512)), full((1, _KEY_DIM)),
            full((8, 3)), full((1, 8)), full((1, 8)), full((1, 1)),
            full((16, 3)), full((1, 16)), full((1, 16)), full((1, 1)),
            full((_PFC_DIM, _D_MEM)), full((1, _PFC_DIM)),
            full((12, _D_MEM)), full((1, 12)),
            pl.BlockSpec(memory_space=pl.ANY),
            pl.BlockSpec(memory_space=pl.ANY),
            pl.BlockSpec(memory_space=pl.ANY),
        ],
        out_specs=[full((1, _PFC_DIM)), full((1, 1)), full((1, 12)),
                   full((64, 128))],
        out_shape=[
            jax.ShapeDtypeStruct((1, _PFC_DIM), jnp.float32),
            jax.ShapeDtypeStruct((1, 1), jnp.float32),
            jax.ShapeDtypeStruct((1, 12), jnp.float32),
            jax.ShapeDtypeStruct((64, 128), jnp.float32),
        ],
        scratch_shapes=[
            pltpu.VMEM((_NBLK, _BLK), jnp.float32),
            pltpu.VMEM((1, _KEY_DIM), jnp.float32),
            pltpu.VMEM((_EPS, _D_MEM), jnp.float32),
            pltpu.VMEM((1, _EPS), jnp.float32),
            pltpu.VMEM((1, _EPS), jnp.float32),
            pltpu.SemaphoreType.DMA((3,)),
        ],
        compiler_params=pltpu.CompilerParams(
            dimension_semantics=("arbitrary",)),
    )(prototypes, act, pfc_state, ctd, kp_w1, kp_b1.reshape(1, -1),
      kp_w2, kp_b2.reshape(1, -1), sc_w1, sc_b1.reshape(1, -1),
      sc_w2, sc_b2.reshape(1, 1), g_w1, g_b1.reshape(1, -1),
      g_w2, g_b2.reshape(1, 1), rp_w, rp_b.reshape(1, -1),
      rn_w, rn_b.reshape(1, -1), episodes, ep_td_errors, ep_timestamps)

    o_pfc, o_alpha, o_nm, o_onehot = outs
    return jnp.concatenate([o_pfc.reshape(_PFC_DIM), o_alpha.reshape(1),
                            o_onehot.reshape(_N_SLOTS), o_nm.reshape(12)])
